# Initial kernel scaffold; baseline (speedup 1.0000x reference)
import jax
import jax.numpy as jnp
from jax.experimental import pallas as pl


def kernel(coords, hidden, edges, Wm1, bm1, Wm2, bm2, Wc1, bc1, Wc2, Wh1, bh1, Wh2, bh2):
    raise NotImplementedError("write your pallas kernel here")


# speedup vs baseline: 13.3354x; 13.3354x over previous
import functools

import jax
import jax.numpy as jnp
from jax import lax
from jax.experimental import pallas as pl
from jax.experimental.pallas import tpu as pltpu
from jax.experimental.pallas import tpu_sc as plsc

F32 = jnp.float32
B, N, H, M = 8, 64, 256, 128
NN = B * N
NC, NS = 2, 16
NW = NC * NS


def _silu(x):
    return x * jax.nn.sigmoid(x)


def _prep_body(hf, cpad, srcf, dstf, ws, wd, ts, td, wv, nnb):
    ps = jnp.dot(hf[...], ws[...], preferred_element_type=F32)
    pd = jnp.dot(hf[...], wd[...], preferred_element_type=F32)
    cp = cpad[...]
    ts[:, 0, :] = ps
    ts[:, 1, :] = cp
    td[:, 0, :] = pd
    td[:, 1, :] = -cp
    e = srcf.shape[0]
    iota = lax.broadcasted_iota(jnp.int32, (e, N), 1)
    os_ = (srcf[...] == iota).astype(F32)
    od_ = (dstf[...] == iota).astype(F32)
    cnt = lax.dot_general(os_, od_, (((0,), (0,)), ((), ())),
                          preferred_element_type=F32)
    mult = jnp.sum(jnp.dot(os_, cnt, preferred_element_type=F32) * od_,
                   axis=1, keepdims=True)
    wv[...] = 1.0 / mult
    nnb[...] = jnp.sum(cnt, axis=1, keepdims=True)


def _edge_body(gs, gd, wv, wn, bm1, wm2, bm2, wc1, bc1, wc2r, out):
    g = gs[:, 0, :] + gd[:, 0, :]
    diff = gs[:, 1, :] + gd[:, 1, :]
    d2 = jnp.sum(diff * diff, axis=1, keepdims=True)
    n2 = jnp.sqrt(d2)
    m = _silu(g + n2 * wn[...] + bm1[...])
    f = _silu(jnp.dot(m, wm2[...], preferred_element_type=F32) + bm2[...])
    cp = _silu(jnp.dot(f, wc1[...], preferred_element_type=F32) + bc1[...])
    c = jnp.sum(cp * wc2r[...], axis=1, keepdims=True)
    w = wv[...]
    out[:, 0, :] = f * w
    out[:, 1, :] = diff * (c * w)


def _node_body(hf, cpad, part, nnb, wh1h, wh1s, bh1, wh2, bh2, co, ho):
    s = part[0] + part[1]
    sum_h = s[:, 0, :]
    sum_t = s[:, 1, :]
    co[...] = cpad[...] + sum_t / nnb[...]
    pre = _silu(jnp.dot(hf[...], wh1h[...], preferred_element_type=F32)
                + jnp.dot(sum_h, wh1s[...], preferred_element_type=F32)
                + bh1[...])
    ho[...] = jnp.dot(pre, wh2[...], preferred_element_type=F32) + bh2[...]


def _mesh():
    return plsc.VectorSubcoreMesh(core_axis_name="c", subcore_axis_name="s",
                                  num_cores=NC, num_subcores=NS)


def _sc_gather(ts, td, srcr, dstr, ef):
    chunk = ef // NW
    nj = chunk // 128

    @functools.partial(
        pl.kernel, mesh=_mesh(),
        out_type=(jax.ShapeDtypeStruct((ef, 2, 128), F32),
                  jax.ShapeDtypeStruct((ef, 2, 128), F32)),
        scratch_types=[pltpu.VMEM((nj, 128), jnp.int32),
                       pltpu.VMEM((nj, 128), jnp.int32),
                       pltpu.VMEM((128, 2, 128), F32),
                       pltpu.VMEM((128, 2, 128), F32),
                       pltpu.SemaphoreType.DMA,
                       pltpu.SemaphoreType.DMA],
    )
    def k(ts_hbm, td_hbm, src_hbm, dst_hbm, gs_hbm, gd_hbm,
          idxs, idxd, bufs, bufd, sem_s, sem_d):
        cid = lax.axis_index("c")
        sid = lax.axis_index("s")
        wid = sid * NC + cid
        base = wid * chunk
        pltpu.sync_copy(src_hbm.at[wid], idxs)
        pltpu.sync_copy(dst_hbm.at[wid], idxd)
        for j in range(nj):
            cs = pltpu.async_copy(ts_hbm.at[idxs.at[j]], bufs, sem_s)
            cd = pltpu.async_copy(td_hbm.at[idxd.at[j]], bufd, sem_d)
            cs.wait()
            cd.wait()
            pltpu.sync_copy(bufs, gs_hbm.at[pl.ds(base + j * 128, 128)])
            pltpu.sync_copy(bufd, gd_hbm.at[pl.ds(base + j * 128, 128)])

    return k(ts, td, srcr, dstr)


def _sc_scatter(rows, dstr, zeros, ef):
    chunk = ef // NW
    nj = chunk // 128

    @functools.partial(
        pl.kernel, mesh=_mesh(),
        out_type=jax.ShapeDtypeStruct((NC, NN, 2, 128), F32),
        scratch_types=[pltpu.VMEM((nj, 128), jnp.int32),
                       pltpu.VMEM((chunk, 2, 128), F32),
                       pltpu.VMEM_SHARED((NN, 2, 128), F32)],
    )
    def k(r_hbm, dst_hbm, z_hbm, out_hbm, idx, buf, acc):
        cid = lax.axis_index("c")
        sid = lax.axis_index("s")
        wid = sid * NC + cid
        base = wid * chunk

        @pl.when(sid == 0)
        def _():
            pltpu.sync_copy(z_hbm, acc)

        plsc.subcore_barrier()
        pltpu.sync_copy(dst_hbm.at[wid], idx)
        pltpu.sync_copy(r_hbm.at[pl.ds(base, chunk)], buf)
        for j in range(nj):
            pltpu.sync_copy(buf.at[pl.ds(j * 128, 128)],
                            acc.at[idx.at[j]], add=True)
        plsc.subcore_barrier()

        @pl.when(sid == 0)
        def _():
            pltpu.sync_copy(acc, out_hbm.at[cid])

    return k(rows, dstr, zeros)


def kernel(coords, hidden, edges, Wm1, bm1, Wm2, bm2, Wc1, bc1, Wc2,
           Wh1, bh1, Wh2, bh2):
    e = edges.shape[1]
    ef = B * e
    cf = coords.reshape(NN, 3).astype(F32)
    hf = hidden.reshape(NN, H).astype(F32)
    cpad = jnp.concatenate([cf, jnp.zeros((NN, 125), F32)], axis=1)

    srcf = edges[0].astype(jnp.int32)[:, None]
    dstf = edges[1].astype(jnp.int32)[:, None]

    ts, td, wv, nnb = pl.pallas_call(
        _prep_body,
        out_shape=(jax.ShapeDtypeStruct((NN, 2, 128), F32),
                   jax.ShapeDtypeStruct((NN, 2, 128), F32),
                   jax.ShapeDtypeStruct((e, 1), F32),
                   jax.ShapeDtypeStruct((N, 1), F32)),
    )(hf, cpad, srcf, dstf, Wm1[1:1 + H], Wm1[1 + H:])

    offs = (jnp.arange(B, dtype=jnp.int32) * N)[:, None]
    src_flat = (edges[0][None, :] + offs).reshape(NW, ef // NW // 128, 128)
    dst_flat = (edges[1][None, :] + offs).reshape(NW, ef // NW // 128, 128)

    gs, gd = _sc_gather(ts, td, src_flat, dst_flat, ef)

    w_flat = jnp.tile(wv, (B, 1))
    rblk = 1024
    grid = ef // rblk
    full = lambda shape: pl.BlockSpec(shape, lambda i: tuple(0 for _ in shape))
    rows = pl.pallas_call(
        _edge_body,
        grid=(grid,),
        in_specs=[pl.BlockSpec((rblk, 2, 128), lambda i: (i, 0, 0)),
                  pl.BlockSpec((rblk, 2, 128), lambda i: (i, 0, 0)),
                  pl.BlockSpec((rblk, 1), lambda i: (i, 0)),
                  full((1, M)), full((1, M)), full((M, M)), full((1, M)),
                  full((M, M)), full((1, M)), full((1, M))],
        out_specs=pl.BlockSpec((rblk, 2, 128), lambda i: (i, 0, 0)),
        out_shape=jax.ShapeDtypeStruct((ef, 2, 128), F32),
    )(gs, gd, w_flat, Wm1[0:1], bm1[None, :], Wm2, bm2[None, :],
      Wc1, bc1[None, :], Wc2.reshape(1, M))

    zeros = jnp.zeros((NN, 2, 128), F32)
    partials = _sc_scatter(rows, dst_flat, zeros, ef)

    nnb_flat = jnp.tile(nnb, (B, 1))
    co, ho = pl.pallas_call(
        _node_body,
        out_shape=(jax.ShapeDtypeStruct((NN, 128), F32),
                   jax.ShapeDtypeStruct((NN, H), F32)),
    )(hf, cpad, partials, nnb_flat, Wh1[:H], Wh1[H:],
      bh1[None, :], Wh2, bh2[None, :])

    coords_out = co[:, :3].reshape(B, N, 3)
    hidden_out = ho.reshape(B, N, H)
    return coords_out, hidden_out


# speedup vs baseline: 20.5206x; 1.5388x over previous
import functools

import jax
import jax.numpy as jnp
from jax import lax
from jax.experimental import pallas as pl
from jax.experimental.pallas import tpu as pltpu
from jax.experimental.pallas import tpu_sc as plsc

F32 = jnp.float32
B, N, H, M = 8, 64, 256, 128
NN = B * N
NC, NS = 2, 16
NW = NC * NS


def _silu(x):
    return x * jax.nn.sigmoid(x)


def _prep_body(hf, srcf, dstf, ws, wd, ts, td, wv, nnb):
    ts[...] = jnp.dot(hf[...], ws[...], preferred_element_type=F32)
    td[...] = jnp.dot(hf[...], wd[...], preferred_element_type=F32)
    e = srcf.shape[0]
    iota = lax.broadcasted_iota(jnp.int32, (e, N), 1)
    os_ = (srcf[...] == iota).astype(F32)
    od_ = (dstf[...] == iota).astype(F32)
    cnt = lax.dot_general(os_, od_, (((0,), (0,)), ((), ())),
                          preferred_element_type=F32)
    mult = jnp.sum(jnp.dot(os_, cnt, preferred_element_type=F32) * od_,
                   axis=1, keepdims=True)
    wv[...] = 1.0 / mult
    nnb[...] = jnp.sum(cnt, axis=1, keepdims=True)


def _edge_body(gs, gd, srcf, dstf, cpad, wv, wn, bm1, wm2, bm2, wc1, bc1,
               wc2r, rout, sumt):
    e = srcf.shape[0]
    iota = lax.broadcasted_iota(jnp.int32, (e, N), 1)
    os_ = (srcf[...] == iota).astype(F32)
    od_ = (dstf[...] == iota).astype(F32)
    cp = cpad[...]
    diff = (jnp.dot(os_, cp, preferred_element_type=F32)
            - jnp.dot(od_, cp, preferred_element_type=F32))
    d2 = jnp.sum(diff * diff, axis=1, keepdims=True)
    n2 = jnp.sqrt(d2)
    g = gs[...] + gd[...]
    m = _silu(g + n2 * wn[...] + bm1[...])
    f = _silu(jnp.dot(m, wm2[...], preferred_element_type=F32) + bm2[...])
    cp_ = _silu(jnp.dot(f, wc1[...], preferred_element_type=F32) + bc1[...])
    c = jnp.sum(cp_ * wc2r[...], axis=1, keepdims=True)
    w = wv[...]
    rout[...] = f * w
    sumt[...] = lax.dot_general(od_, diff * (c * w),
                                (((0,), (0,)), ((), ())),
                                preferred_element_type=F32)


def _node_body(hf, cpad, part, sumt, nnb, wh1h, wh1s, bh1, wh2, bh2, co, ho):
    sum_h = part[0] + part[1]
    co[...] = cpad[...] + sumt[...] / nnb[...]
    pre = _silu(jnp.dot(hf[...], wh1h[...], preferred_element_type=F32)
                + jnp.dot(sum_h, wh1s[...], preferred_element_type=F32)
                + bh1[...])
    ho[...] = jnp.dot(pre, wh2[...], preferred_element_type=F32) + bh2[...]


def _mesh():
    return plsc.VectorSubcoreMesh(core_axis_name="c", subcore_axis_name="s",
                                  num_cores=NC, num_subcores=NS)


def _sc_gather(ts, td, srcr, dstr, ef):
    chunk = ef // NW
    nj = chunk // 128

    @functools.partial(
        pl.kernel, mesh=_mesh(),
        out_type=(jax.ShapeDtypeStruct((ef, M), F32),
                  jax.ShapeDtypeStruct((ef, M), F32)),
        scratch_types=[pltpu.VMEM((nj, 128), jnp.int32),
                       pltpu.VMEM((nj, 128), jnp.int32),
                       pltpu.VMEM((chunk, M), F32),
                       pltpu.VMEM((chunk, M), F32),
                       pltpu.SemaphoreType.DMA,
                       pltpu.SemaphoreType.DMA],
    )
    def k(ts_hbm, td_hbm, src_hbm, dst_hbm, gs_hbm, gd_hbm,
          idxs, idxd, bufs, bufd, sem_s, sem_d):
        cid = lax.axis_index("c")
        sid = lax.axis_index("s")
        wid = sid * NC + cid
        base = wid * chunk
        pltpu.sync_copy(src_hbm.at[wid], idxs)
        pltpu.sync_copy(dst_hbm.at[wid], idxd)
        cps = [pltpu.async_copy(ts_hbm.at[idxs.at[j]],
                                bufs.at[pl.ds(j * 128, 128)], sem_s)
               for j in range(nj)]
        cpd = [pltpu.async_copy(td_hbm.at[idxd.at[j]],
                                bufd.at[pl.ds(j * 128, 128)], sem_d)
               for j in range(nj)]
        for c_ in cps:
            c_.wait()
        for c_ in cpd:
            c_.wait()
        pltpu.sync_copy(bufs, gs_hbm.at[pl.ds(base, chunk)])
        pltpu.sync_copy(bufd, gd_hbm.at[pl.ds(base, chunk)])

    return k(ts, td, srcr, dstr)


def _sc_scatter(rows, dstr, zeros, ef):
    chunk = ef // NW
    nj = chunk // 128

    @functools.partial(
        pl.kernel, mesh=_mesh(),
        out_type=jax.ShapeDtypeStruct((NC, NN, M), F32),
        scratch_types=[pltpu.VMEM((nj, 128), jnp.int32),
                       pltpu.VMEM((chunk, M), F32),
                       pltpu.VMEM_SHARED((NN, M), F32)],
    )
    def k(r_hbm, dst_hbm, z_hbm, out_hbm, idx, buf, acc):
        cid = lax.axis_index("c")
        sid = lax.axis_index("s")
        wid = sid * NC + cid
        base = wid * chunk

        @pl.when(sid == 0)
        def _():
            pltpu.sync_copy(z_hbm, acc)

        plsc.subcore_barrier()
        pltpu.sync_copy(dst_hbm.at[wid], idx)
        pltpu.sync_copy(r_hbm.at[pl.ds(base, chunk)], buf)
        for j in range(nj):
            pltpu.sync_copy(buf.at[pl.ds(j * 128, 128)],
                            acc.at[idx.at[j]], add=True)
        plsc.subcore_barrier()

        @pl.when(sid == 0)
        def _():
            pltpu.sync_copy(acc, out_hbm.at[cid])

    return k(rows, dstr, zeros)


def kernel(coords, hidden, edges, Wm1, bm1, Wm2, bm2, Wc1, bc1, Wc2,
           Wh1, bh1, Wh2, bh2):
    e = edges.shape[1]
    ef = B * e
    cf = coords.reshape(NN, 3).astype(F32)
    hf = hidden.reshape(NN, H).astype(F32)
    cpad = jnp.concatenate([cf, jnp.zeros((NN, 125), F32)], axis=1)

    srcf = edges[0].astype(jnp.int32)[:, None]
    dstf = edges[1].astype(jnp.int32)[:, None]

    ts, td, wv, nnb = pl.pallas_call(
        _prep_body,
        out_shape=(jax.ShapeDtypeStruct((NN, M), F32),
                   jax.ShapeDtypeStruct((NN, M), F32),
                   jax.ShapeDtypeStruct((e, 1), F32),
                   jax.ShapeDtypeStruct((N, 1), F32)),
    )(hf, srcf, dstf, Wm1[1:1 + H], Wm1[1 + H:])

    offs = (jnp.arange(B, dtype=jnp.int32) * N)[:, None]
    src_flat = (edges[0][None, :] + offs).reshape(NW, ef // NW // 128, 128)
    dst_flat = (edges[1][None, :] + offs).reshape(NW, ef // NW // 128, 128)

    gs, gd = _sc_gather(ts, td, src_flat, dst_flat, ef)

    full = lambda shape: pl.BlockSpec(shape, lambda i: tuple(0 for _ in shape))
    rows, sumt = pl.pallas_call(
        _edge_body,
        grid=(B,),
        in_specs=[pl.BlockSpec((e, M), lambda i: (i, 0)),
                  pl.BlockSpec((e, M), lambda i: (i, 0)),
                  full((e, 1)), full((e, 1)),
                  pl.BlockSpec((N, 128), lambda i: (i, 0)),
                  full((e, 1)),
                  full((1, M)), full((1, M)), full((M, M)), full((1, M)),
                  full((M, M)), full((1, M)), full((1, M))],
        out_specs=(pl.BlockSpec((e, M), lambda i: (i, 0)),
                   pl.BlockSpec((N, 128), lambda i: (i, 0))),
        out_shape=(jax.ShapeDtypeStruct((ef, M), F32),
                   jax.ShapeDtypeStruct((NN, 128), F32)),
    )(gs, gd, srcf, dstf, cpad, wv, Wm1[0:1], bm1[None, :], Wm2,
      bm2[None, :], Wc1, bc1[None, :], Wc2.reshape(1, M))

    zeros = jnp.zeros((NN, M), F32)
    partials = _sc_scatter(rows, dst_flat, zeros, ef)

    nnb_flat = jnp.tile(nnb, (B, 1))
    co, ho = pl.pallas_call(
        _node_body,
        out_shape=(jax.ShapeDtypeStruct((NN, 128), F32),
                   jax.ShapeDtypeStruct((NN, H), F32)),
    )(hf, cpad, partials, sumt, nnb_flat, Wh1[:H], Wh1[H:],
      bh1[None, :], Wh2, bh2[None, :])

    coords_out = co[:, :3].reshape(B, N, 3)
    hidden_out = ho.reshape(B, N, H)
    return coords_out, hidden_out


# speedup vs baseline: 23.3184x; 1.1363x over previous
import functools

import jax
import jax.numpy as jnp
from jax import lax
from jax.experimental import pallas as pl
from jax.experimental.pallas import tpu as pltpu
from jax.experimental.pallas import tpu_sc as plsc

F32 = jnp.float32
B, N, H, M = 8, 64, 256, 128
NN = B * N
NC, NS = 2, 16
NW = NC * NS


def _silu(x):
    return x * jax.nn.sigmoid(x)


def _prep_body(hf, srcf, dstf, ws, wd, ts, td, wv, nnb):
    ts[...] = jnp.dot(hf[...], ws[...], preferred_element_type=F32)
    td[...] = jnp.dot(hf[...], wd[...], preferred_element_type=F32)
    e = srcf.shape[0]
    iota = lax.broadcasted_iota(jnp.int32, (e, N), 1)
    os_ = (srcf[...] == iota).astype(F32)
    od_ = (dstf[...] == iota).astype(F32)
    cnt = lax.dot_general(os_, od_, (((0,), (0,)), ((), ())),
                          preferred_element_type=F32)
    mult = jnp.sum(jnp.dot(os_, cnt, preferred_element_type=F32) * od_,
                   axis=1, keepdims=True)
    wv[...] = 1.0 / mult
    nnb[...] = jnp.sum(cnt, axis=1, keepdims=True)


def _edge_body(gs, gd, srcf, dstf, cpad, hf, wv, nnb, wn, bm1, wm2, bm2,
               wc1, bc1, wc2r, wh1h, wh1s, bh1, wh2, bh2, co, ho):
    e = srcf.shape[0]
    iota = lax.broadcasted_iota(jnp.int32, (e, N), 1)
    os_ = (srcf[...] == iota).astype(F32)
    od_ = (dstf[...] == iota).astype(F32)
    cp = cpad[...]
    diff = (jnp.dot(os_, cp, preferred_element_type=F32)
            - jnp.dot(od_, cp, preferred_element_type=F32))
    d2 = jnp.sum(diff * diff, axis=1, keepdims=True)
    n2 = jnp.sqrt(d2)
    g = gs[...] + gd[...]
    m = _silu(g + n2 * wn[...] + bm1[...])
    f = _silu(jnp.dot(m, wm2[...], preferred_element_type=F32) + bm2[...])
    cq = _silu(jnp.dot(f, wc1[...], preferred_element_type=F32) + bc1[...])
    c = jnp.sum(cq * wc2r[...], axis=1, keepdims=True)
    w = wv[...]
    sum_h = lax.dot_general(od_, f * w, (((0,), (0,)), ((), ())),
                            preferred_element_type=F32)
    sum_t = lax.dot_general(od_, diff * (c * w), (((0,), (0,)), ((), ())),
                            preferred_element_type=F32)
    co[...] = cp + sum_t / nnb[...]
    pre = _silu(jnp.dot(hf[...], wh1h[...], preferred_element_type=F32)
                + jnp.dot(sum_h, wh1s[...], preferred_element_type=F32)
                + bh1[...])
    ho[...] = jnp.dot(pre, wh2[...], preferred_element_type=F32) + bh2[...]


def _mesh():
    return plsc.VectorSubcoreMesh(core_axis_name="c", subcore_axis_name="s",
                                  num_cores=NC, num_subcores=NS)


def _sc_gather(ts, td, srcr, dstr, ef):
    chunk = ef // NW
    nj = chunk // 128

    @functools.partial(
        pl.kernel, mesh=_mesh(),
        out_type=(jax.ShapeDtypeStruct((ef, M), F32),
                  jax.ShapeDtypeStruct((ef, M), F32)),
        scratch_types=[pltpu.VMEM((nj, 128), jnp.int32),
                       pltpu.VMEM((nj, 128), jnp.int32),
                       pltpu.VMEM((chunk, M), F32),
                       pltpu.VMEM((chunk, M), F32),
                       pltpu.SemaphoreType.DMA,
                       pltpu.SemaphoreType.DMA],
    )
    def k(ts_hbm, td_hbm, src_hbm, dst_hbm, gs_hbm, gd_hbm,
          idxs, idxd, bufs, bufd, sem_s, sem_d):
        cid = lax.axis_index("c")
        sid = lax.axis_index("s")
        wid = sid * NC + cid
        base = wid * chunk
        pltpu.sync_copy(src_hbm.at[wid], idxs)
        pltpu.sync_copy(dst_hbm.at[wid], idxd)
        cps = [pltpu.async_copy(ts_hbm.at[idxs.at[j]],
                                bufs.at[pl.ds(j * 128, 128)], sem_s)
               for j in range(nj)]
        cpd = [pltpu.async_copy(td_hbm.at[idxd.at[j]],
                                bufd.at[pl.ds(j * 128, 128)], sem_d)
               for j in range(nj)]
        for c_ in cps:
            c_.wait()
        for c_ in cpd:
            c_.wait()
        pltpu.sync_copy(bufs, gs_hbm.at[pl.ds(base, chunk)])
        pltpu.sync_copy(bufd, gd_hbm.at[pl.ds(base, chunk)])

    return k(ts, td, srcr, dstr)


def kernel(coords, hidden, edges, Wm1, bm1, Wm2, bm2, Wc1, bc1, Wc2,
           Wh1, bh1, Wh2, bh2):
    e = edges.shape[1]
    ef = B * e
    cf = coords.reshape(NN, 3).astype(F32)
    hf = hidden.reshape(NN, H).astype(F32)
    cpad = jnp.concatenate([cf, jnp.zeros((NN, 125), F32)], axis=1)

    srcf = edges[0].astype(jnp.int32)[:, None]
    dstf = edges[1].astype(jnp.int32)[:, None]

    ts, td, wv, nnb = pl.pallas_call(
        _prep_body,
        out_shape=(jax.ShapeDtypeStruct((NN, M), F32),
                   jax.ShapeDtypeStruct((NN, M), F32),
                   jax.ShapeDtypeStruct((e, 1), F32),
                   jax.ShapeDtypeStruct((N, 1), F32)),
    )(hf, srcf, dstf, Wm1[1:1 + H], Wm1[1 + H:])

    offs = (jnp.arange(B, dtype=jnp.int32) * N)[:, None]
    src_flat = (edges[0][None, :] + offs).reshape(NW, ef // NW // 128, 128)
    dst_flat = (edges[1][None, :] + offs).reshape(NW, ef // NW // 128, 128)

    gs, gd = _sc_gather(ts, td, src_flat, dst_flat, ef)

    full = lambda shape: pl.BlockSpec(shape, lambda i: tuple(0 for _ in shape))
    co, ho = pl.pallas_call(
        _edge_body,
        grid=(B,),
        in_specs=[pl.BlockSpec((e, M), lambda i: (i, 0)),
                  pl.BlockSpec((e, M), lambda i: (i, 0)),
                  full((e, 1)), full((e, 1)),
                  pl.BlockSpec((N, 128), lambda i: (i, 0)),
                  pl.BlockSpec((N, H), lambda i: (i, 0)),
                  full((e, 1)), full((N, 1)),
                  full((1, M)), full((1, M)), full((M, M)), full((1, M)),
                  full((M, M)), full((1, M)), full((1, M)),
                  full((H, M)), full((M, M)), full((1, M)),
                  full((M, H)), full((1, H))],
        out_specs=(pl.BlockSpec((N, 128), lambda i: (i, 0)),
                   pl.BlockSpec((N, H), lambda i: (i, 0))),
        out_shape=(jax.ShapeDtypeStruct((NN, 128), F32),
                   jax.ShapeDtypeStruct((NN, H), F32)),
    )(gs, gd, srcf, dstf, cpad, hf, wv, nnb, Wm1[0:1], bm1[None, :], Wm2,
      bm2[None, :], Wc1, bc1[None, :], Wc2.reshape(1, M),
      Wh1[:H], Wh1[H:], bh1[None, :], Wh2, bh2[None, :])

    coords_out = co[:, :3].reshape(B, N, 3)
    hidden_out = ho.reshape(B, N, H)
    return coords_out, hidden_out
